# baseline (speedup 1.0000x reference)
import dataclasses
import functools

import jax
import jax.numpy as jnp
from jax import lax
from jax.experimental import pallas as pl
from jax.experimental.pallas import tpu as pltpu
from jax.experimental.pallas import tpu_sc as plsc

N = 10000
E = 320000
D_IN = 128
D_H = 256
D_OUT = 128
G = 64

NP = 10240
ROWS_PER_SUB = NP // 16
BR = 2048
NBLK = NP // BR
EK = 80

_mesh = plsc.VectorSubcoreMesh(
    core_axis_name="c", subcore_axis_name="s", num_cores=2, num_subcores=16)

_f32 = jnp.float32
_HIGH = jax.lax.Precision.HIGHEST

_sc_params = pltpu.CompilerParams()
if "needs_layout_passes" in pltpu.CompilerParams.__dataclass_fields__:
  _sc_params = dataclasses.replace(_sc_params, needs_layout_passes=False)


def _dot(a, b):
  return jnp.dot(a, b, preferred_element_type=_f32)


def _deg_body(dst_hbm, outa_hbm, outb_hbm, hist, dchunk, accv, tmpv, hists,
              sem):
  del sem
  c = lax.axis_index("c")
  s = lax.axis_index("s")
  w = c * 16 + s
  per_w = E // 32
  zero16 = jnp.zeros((16,), _f32)
  one16 = jnp.ones((16,), _f32)

  @pl.loop(0, NP, step=16)
  def _(i):
    hist[pl.ds(i, 16)] = zero16

  base = w * per_w

  @pl.loop(0, per_w, step=2000)
  def _(i):
    pltpu.sync_copy(dst_hbm.at[pl.ds(base + i, 2000)], dchunk)

    @pl.loop(0, 2000, step=16)
    def _(j):
      idx = dchunk[pl.ds(j, 16)]
      plsc.addupdate_scatter(hist, [idx], one16)

  pltpu.sync_copy(hist, hists.at[s])
  plsc.subcore_barrier()

  cbase = s * ROWS_PER_SUB
  pltpu.sync_copy(hists.at[0, pl.ds(cbase, ROWS_PER_SUB)], accv)

  @pl.loop(1, 16)
  def _(r):
    pltpu.sync_copy(hists.at[r, pl.ds(cbase, ROWS_PER_SUB)], tmpv)

    @pl.loop(0, ROWS_PER_SUB, step=16)
    def _(j):
      accv[pl.ds(j, 16)] = accv[pl.ds(j, 16)] + tmpv[pl.ds(j, 16)]

  @pl.when(c == 0)
  def _():
    pltpu.sync_copy(accv, outa_hbm.at[pl.ds(cbase, ROWS_PER_SUB)])

  @pl.when(c == 1)
  def _():
    pltpu.sync_copy(accv, outb_hbm.at[pl.ds(cbase, ROWS_PER_SUB)])


@jax.jit
def _sc_degree(dst):
  return pl.kernel(
      _deg_body,
      out_type=[jax.ShapeDtypeStruct((NP,), _f32),
                jax.ShapeDtypeStruct((NP,), _f32)],
      mesh=_mesh,
      scratch_types=[
          pltpu.VMEM((NP,), _f32),
          pltpu.VMEM((2000,), jnp.int32),
          pltpu.VMEM((ROWS_PER_SUB,), _f32),
          pltpu.VMEM((ROWS_PER_SUB,), _f32),
          pltpu.VMEM_SHARED((16, NP), _f32),
          pltpu.SemaphoreType.DMA,
      ],
      compiler_params=_sc_params,
  )(dst)


CHUNK = 80
DEPTH = 4
IDXB = 16


def _agg_pipelined(table_hbm, acc, bufs, ssems, src_hbm, dst_hbm, wsel,
                   sidx, didx, nchunks):
  rows_l = [r for r, _ in bufs]
  gsems = [g for _, g in bufs]

  def g_wait(b):
    pltpu.make_async_copy(table_hbm.at[sidx.at[0]], rows_l[b], gsems[b]).wait()

  def s_wait(b):
    pltpu.make_async_copy(rows_l[b], acc.at[didx.at[0]], ssems[b]).wait()

  @pl.loop(0, nchunks, step=IDXB)
  def _(blk):
    pltpu.sync_copy(src_hbm.at[wsel, pl.ds(blk, IDXB)], sidx)
    pltpu.sync_copy(dst_hbm.at[wsel, pl.ds(blk, IDXB)], didx)
    pltpu.async_copy(table_hbm.at[sidx.at[0]], rows_l[0], gsems[0])
    pltpu.async_copy(table_hbm.at[sidx.at[1]], rows_l[1], gsems[1])

    @pl.loop(0, IDXB, step=DEPTH)
    def _(k):
      for b in range(DEPTH):
        j = k + b
        bn = (b + 2) % DEPTH

        @pl.when(j >= 2)
        def _():
          s_wait(bn)

        @pl.when(j + 2 < IDXB)
        def _():
          pltpu.async_copy(table_hbm.at[sidx.at[j + 2]], rows_l[bn],
                           gsems[bn])

        g_wait(b)
        pltpu.async_copy(rows_l[b], acc.at[didx.at[j]], ssems[b], add=True)

    s_wait((IDXB - 2) % DEPTH)
    s_wait((IDXB - 1) % DEPTH)


def _agg1_body(xs_hbm, zeros_hbm, src_hbm, dst_hbm, outa_hbm, outb_hbm,
               acc, r0, r1, r2, r3, sidx, didx, s0, s1, s2, s3,
               t0, t1, t2, t3):
  c = lax.axis_index("c")
  s = lax.axis_index("s")
  rbase = s * ROWS_PER_SUB
  w = c * 16 + s

  @pl.when(c == 0)
  def _():
    pltpu.sync_copy(xs_hbm.at[pl.ds(rbase, ROWS_PER_SUB)],
                    acc.at[pl.ds(rbase, ROWS_PER_SUB)])

  @pl.when(c == 1)
  def _():
    pltpu.sync_copy(zeros_hbm.at[pl.ds(rbase, ROWS_PER_SUB)],
                    acc.at[pl.ds(rbase, ROWS_PER_SUB)])

  plsc.subcore_barrier()
  _agg_pipelined(xs_hbm, acc, ((r0, s0), (r1, s1), (r2, s2), (r3, s3)),
                 (t0, t1, t2, t3), src_hbm, dst_hbm, w, sidx, didx,
                 src_hbm.shape[1])
  plsc.subcore_barrier()

  @pl.when(c == 0)
  def _():
    pltpu.sync_copy(acc.at[pl.ds(rbase, ROWS_PER_SUB)],
                    outa_hbm.at[pl.ds(rbase, ROWS_PER_SUB)])

  @pl.when(c == 1)
  def _():
    pltpu.sync_copy(acc.at[pl.ds(rbase, ROWS_PER_SUB)],
                    outb_hbm.at[pl.ds(rbase, ROWS_PER_SUB)])


@jax.jit
def _sc_agg1(xs, zeros, src, dst):
  return pl.kernel(
      _agg1_body,
      out_type=[jax.ShapeDtypeStruct((NP, D_IN), _f32),
                jax.ShapeDtypeStruct((NP, D_IN), _f32)],
      mesh=_mesh,
      scratch_types=[
          pltpu.VMEM_SHARED((NP, D_IN), _f32),
          pltpu.VMEM((CHUNK, D_IN), _f32),
          pltpu.VMEM((CHUNK, D_IN), _f32),
          pltpu.VMEM((CHUNK, D_IN), _f32),
          pltpu.VMEM((CHUNK, D_IN), _f32),
          pltpu.VMEM((IDXB, CHUNK), jnp.int32),
          pltpu.VMEM((IDXB, CHUNK), jnp.int32),
      ] + [pltpu.SemaphoreType.DMA] * 8,
  )(xs, zeros, src, dst)


def _agg2_body(ha_hbm, hb_hbm, src_hbm, dst_hbm, outa_hbm, outb_hbm,
               acc, r0, r1, r2, r3, sidx, didx, s0, s1, s2, s3,
               t0, t1, t2, t3):
  c = lax.axis_index("c")
  s = lax.axis_index("s")
  rbase = s * ROWS_PER_SUB

  def run(table_hbm, out_hbm):
    pltpu.sync_copy(table_hbm.at[pl.ds(rbase, ROWS_PER_SUB)],
                    acc.at[pl.ds(rbase, ROWS_PER_SUB)])
    plsc.subcore_barrier()
    _agg_pipelined(table_hbm, acc, ((r0, s0), (r1, s1), (r2, s2), (r3, s3)),
                   (t0, t1, t2, t3), src_hbm, dst_hbm, s, sidx, didx,
                   src_hbm.shape[1])
    plsc.subcore_barrier()
    pltpu.sync_copy(acc.at[pl.ds(rbase, ROWS_PER_SUB)],
                    out_hbm.at[pl.ds(rbase, ROWS_PER_SUB)])

  @pl.when(c == 0)
  def _():
    run(ha_hbm, outa_hbm)

  @pl.when(c == 1)
  def _():
    run(hb_hbm, outb_hbm)


@jax.jit
def _sc_agg2(ha, hb, src, dst):
  return pl.kernel(
      _agg2_body,
      out_type=[jax.ShapeDtypeStruct((NP, 128), _f32),
                jax.ShapeDtypeStruct((NP, 128), _f32)],
      mesh=_mesh,
      scratch_types=[
          pltpu.VMEM_SHARED((NP, 128), _f32),
          pltpu.VMEM((CHUNK, 128), _f32),
          pltpu.VMEM((CHUNK, 128), _f32),
          pltpu.VMEM((CHUNK, 128), _f32),
          pltpu.VMEM((CHUNK, 128), _f32),
          pltpu.VMEM((IDXB, CHUNK), jnp.int32),
          pltpu.VMEM((IDXB, CHUNK), jnp.int32),
      ] + [pltpu.SemaphoreType.DMA] * 8,
  )(ha, hb, src, dst)


def _tc1_body(dega_ref, degb_ref, x_ref, dinv_ref, xs_ref):
  i = pl.program_id(0)
  deg = dega_ref[...] + degb_ref[...] + 1.0
  dinv = lax.rsqrt(deg)
  row = i * BR + lax.broadcasted_iota(jnp.int32, (BR, 1), 0)
  valid = row < N
  dinv_ref[...] = dinv
  xs_ref[...] = jnp.where(valid, x_ref[...] * dinv, 0.0)


@jax.jit
def _tc_scale(dega, degb, x):
  return pl.pallas_call(
      _tc1_body,
      grid=(NBLK,),
      in_specs=[
          pl.BlockSpec((BR, 1), lambda i: (i, 0)),
          pl.BlockSpec((BR, 1), lambda i: (i, 0)),
          pl.BlockSpec((BR, D_IN), lambda i: (i, 0)),
      ],
      out_specs=[
          pl.BlockSpec((BR, 1), lambda i: (i, 0)),
          pl.BlockSpec((BR, D_IN), lambda i: (i, 0)),
      ],
      out_shape=[
          jax.ShapeDtypeStruct((NP, 1), _f32),
          jax.ShapeDtypeStruct((NP, D_IN), _f32),
      ],
  )(dega, degb, x)


def _tc2_body(p0_ref, p1_ref, dinv_ref, w1_ref, b1_ref, ha_ref, hb_ref):
  agg = p0_ref[...] + p1_ref[...]
  z = _dot(agg, w1_ref[...])
  dinv = dinv_ref[...]
  hs = jnp.maximum(z * dinv + b1_ref[...], 0.0) * dinv
  ha_ref[...] = hs[:, :128]
  hb_ref[...] = hs[:, 128:]


@jax.jit
def _tc_layer1(p0, p1, dinv, w1, b1):
  return pl.pallas_call(
      _tc2_body,
      grid=(NBLK,),
      in_specs=[
          pl.BlockSpec((BR, D_IN), lambda i: (i, 0)),
          pl.BlockSpec((BR, D_IN), lambda i: (i, 0)),
          pl.BlockSpec((BR, 1), lambda i: (i, 0)),
          pl.BlockSpec((D_IN, D_H), lambda i: (0, 0)),
          pl.BlockSpec((1, D_H), lambda i: (0, 0)),
      ],
      out_specs=[
          pl.BlockSpec((BR, 128), lambda i: (i, 0)),
          pl.BlockSpec((BR, 128), lambda i: (i, 0)),
      ],
      out_shape=[
          jax.ShapeDtypeStruct((NP, 128), _f32),
          jax.ShapeDtypeStruct((NP, 128), _f32),
      ],
  )(p0, p1, dinv, w1, b1)


def _tc3_body(qa_ref, qb_ref, dinv_ref, w2_ref, b2_ref, batch_ref,
              wfc_ref, bfc_ref, out_ref, pooled_acc, cnt_acc):
  i = pl.program_id(0)

  @pl.when(i == 0)
  def _():
    pooled_acc[...] = jnp.zeros_like(pooled_acc)
    cnt_acc[...] = jnp.zeros_like(cnt_acc)

  z = _dot(qa_ref[...], w2_ref[:128, :]) + _dot(qb_ref[...], w2_ref[128:, :])
  dinv = dinv_ref[...]
  h2 = jnp.maximum(z * dinv + b2_ref[...], 0.0)
  gids = lax.broadcasted_iota(jnp.int32, (G, BR), 0)
  oht = (batch_ref[0] == gids).astype(_f32)
  pooled_acc[...] += _dot(oht, h2)
  cnt_acc[...] += lax.dot_general(
      oht, jnp.ones((BR, 1), _f32), (((1,), (0,)), ((), ())),
      precision=_HIGH, preferred_element_type=_f32)

  @pl.when(i == NBLK - 1)
  def _():
    pooled = pooled_acc[...] / jnp.maximum(cnt_acc[...], 1.0)
    out_ref[...] = _dot(pooled, wfc_ref[...]) + bfc_ref[...]


@jax.jit
def _tc_layer2_pool(qa, qb, dinv, w2, b2, batch_rows, wfc, bfc):
  return pl.pallas_call(
      _tc3_body,
      grid=(NBLK,),
      in_specs=[
          pl.BlockSpec((BR, 128), lambda i: (i, 0)),
          pl.BlockSpec((BR, 128), lambda i: (i, 0)),
          pl.BlockSpec((BR, 1), lambda i: (i, 0)),
          pl.BlockSpec((D_H, D_H), lambda i: (0, 0)),
          pl.BlockSpec((1, D_H), lambda i: (0, 0)),
          pl.BlockSpec((1, 1, BR), lambda i: (i, 0, 0)),
          pl.BlockSpec((D_H, D_OUT), lambda i: (0, 0)),
          pl.BlockSpec((1, D_OUT), lambda i: (0, 0)),
      ],
      out_specs=pl.BlockSpec((G, D_OUT), lambda i: (0, 0)),
      out_shape=jax.ShapeDtypeStruct((G, D_OUT), _f32),
      scratch_shapes=[
          pltpu.VMEM((G, D_H), _f32),
          pltpu.VMEM((G, 1), _f32),
      ],
  )(qa, qb, dinv, w2, b2, batch_rows, wfc, bfc)


def _pad_chunk_edges(a, nworkers, pad_rows):
  per = E // nworkers
  npad = (-per) % (CHUNK * IDXB)
  a2 = a.reshape(nworkers, per)
  pad = jnp.broadcast_to(pad_rows[None, :npad], (nworkers, npad))
  return jnp.concatenate([a2, pad], axis=1).reshape(
      nworkers, (per + npad) // CHUNK, CHUNK)


def kernel(x, edge_index, batch, W1, b1, W2, b2, Wfc, bfc):
  src = edge_index[0]
  dst = edge_index[1]
  zeros = jnp.zeros((NP, D_IN), _f32)
  batch_rows = jnp.concatenate(
      [batch.astype(jnp.int32), jnp.full((NP - N,), G, jnp.int32)]
  ).reshape(NBLK, 1, BR)

  ar = jnp.arange(2048, dtype=jnp.int32)
  pad_src = (ar * 19) % N
  pad_dst = N + (ar % (NP - N))
  src1 = _pad_chunk_edges(src, 32, pad_src)
  dst1 = _pad_chunk_edges(dst, 32, pad_dst)
  src2 = _pad_chunk_edges(src, 16, pad_src)
  dst2 = _pad_chunk_edges(dst, 16, pad_dst)

  dega, degb = _sc_degree(dst)
  dinv, xs = _tc_scale(dega.reshape(NP, 1), degb.reshape(NP, 1), x)
  p0, p1 = _sc_agg1(xs, zeros, src1, dst1)
  ha, hb = _tc_layer1(p0, p1, dinv, W1, b1.reshape(1, D_H))
  qa, qb = _sc_agg2(ha, hb, src2, dst2)
  return _tc_layer2_pool(qa, qb, dinv, W2, b2.reshape(1, D_H),
                         batch_rows, Wfc, bfc.reshape(1, D_OUT))


# speedup vs baseline: 1.0343x; 1.0343x over previous
import dataclasses
import functools

import jax
import jax.numpy as jnp
from jax import lax
from jax.experimental import pallas as pl
from jax.experimental.pallas import tpu as pltpu
from jax.experimental.pallas import tpu_sc as plsc

N = 10000
E = 320000
D_IN = 128
D_H = 256
D_OUT = 128
G = 64

NP = 10240
ROWS_PER_SUB = NP // 16
BR = 5120
NBLK = NP // BR
EK = 80

_mesh = plsc.VectorSubcoreMesh(
    core_axis_name="c", subcore_axis_name="s", num_cores=2, num_subcores=16)

_f32 = jnp.float32
_HIGH = jax.lax.Precision.HIGHEST

_sc_params = pltpu.CompilerParams()
if "needs_layout_passes" in pltpu.CompilerParams.__dataclass_fields__:
  _sc_params = dataclasses.replace(_sc_params, needs_layout_passes=False)


def _dot(a, b):
  return jnp.dot(a, b, preferred_element_type=_f32)


def _deg_body(dst_hbm, outa_hbm, outb_hbm, hist, dchunk, accv, tmpv, hists,
              sem):
  del sem
  c = lax.axis_index("c")
  s = lax.axis_index("s")
  w = c * 16 + s
  per_w = E // 32
  zero16 = jnp.zeros((16,), _f32)
  one16 = jnp.ones((16,), _f32)

  @pl.loop(0, NP, step=16)
  def _(i):
    hist[pl.ds(i, 16)] = zero16

  base = w * per_w

  @pl.loop(0, per_w, step=2000)
  def _(i):
    pltpu.sync_copy(dst_hbm.at[pl.ds(base + i, 2000)], dchunk)

    @pl.loop(0, 2000, step=16)
    def _(j):
      idx = dchunk[pl.ds(j, 16)]
      plsc.addupdate_scatter(hist, [idx], one16)

  pltpu.sync_copy(hist, hists.at[s])
  plsc.subcore_barrier()

  cbase = s * ROWS_PER_SUB
  pltpu.sync_copy(hists.at[0, pl.ds(cbase, ROWS_PER_SUB)], accv)

  @pl.loop(1, 16)
  def _(r):
    pltpu.sync_copy(hists.at[r, pl.ds(cbase, ROWS_PER_SUB)], tmpv)

    @pl.loop(0, ROWS_PER_SUB, step=16)
    def _(j):
      accv[pl.ds(j, 16)] = accv[pl.ds(j, 16)] + tmpv[pl.ds(j, 16)]

  @pl.when(c == 0)
  def _():
    pltpu.sync_copy(accv, outa_hbm.at[pl.ds(cbase, ROWS_PER_SUB)])

  @pl.when(c == 1)
  def _():
    pltpu.sync_copy(accv, outb_hbm.at[pl.ds(cbase, ROWS_PER_SUB)])


@jax.jit
def _sc_degree(dst):
  return pl.kernel(
      _deg_body,
      out_type=[jax.ShapeDtypeStruct((NP,), _f32),
                jax.ShapeDtypeStruct((NP,), _f32)],
      mesh=_mesh,
      scratch_types=[
          pltpu.VMEM((NP,), _f32),
          pltpu.VMEM((2000,), jnp.int32),
          pltpu.VMEM((ROWS_PER_SUB,), _f32),
          pltpu.VMEM((ROWS_PER_SUB,), _f32),
          pltpu.VMEM_SHARED((16, NP), _f32),
          pltpu.SemaphoreType.DMA,
      ],
      compiler_params=_sc_params,
  )(dst)


CHUNK = 80
DEPTH = 4
IDXB = 16


def _agg_pipelined(table_hbm, acc, bufs, ssems, src_hbm, dst_hbm, wsel,
                   sidx, didx, nchunks):
  del ssems

  @pl.loop(0, nchunks, step=IDXB)
  def _(blk):
    pltpu.sync_copy(src_hbm.at[wsel, pl.ds(blk, IDXB)], sidx)
    pltpu.sync_copy(dst_hbm.at[wsel, pl.ds(blk, IDXB)], didx)
    for b, (rows, sem) in enumerate(bufs):
      pltpu.async_copy(table_hbm.at[sidx.at[b]], rows, sem)

    @pl.loop(0, IDXB, step=DEPTH)
    def _(k):
      for b, (rows, sem) in enumerate(bufs):
        j = k + b
        pltpu.make_async_copy(table_hbm.at[sidx.at[0]], rows, sem).wait()
        pltpu.sync_copy(rows, acc.at[didx.at[j]], add=True)

        @pl.when(j + DEPTH < IDXB)
        def _():
          pltpu.async_copy(table_hbm.at[sidx.at[j + DEPTH]], rows, sem)


def _agg1_body(xs_hbm, zeros_hbm, src_hbm, dst_hbm, outa_hbm, outb_hbm,
               acc, r0, r1, r2, r3, sidx, didx, s0, s1, s2, s3,
               t0, t1, t2, t3):
  c = lax.axis_index("c")
  s = lax.axis_index("s")
  rbase = s * ROWS_PER_SUB
  w = c * 16 + s

  @pl.when(c == 0)
  def _():
    pltpu.sync_copy(xs_hbm.at[pl.ds(rbase, ROWS_PER_SUB)],
                    acc.at[pl.ds(rbase, ROWS_PER_SUB)])

  @pl.when(c == 1)
  def _():
    pltpu.sync_copy(zeros_hbm.at[pl.ds(rbase, ROWS_PER_SUB)],
                    acc.at[pl.ds(rbase, ROWS_PER_SUB)])

  plsc.subcore_barrier()
  _agg_pipelined(xs_hbm, acc, ((r0, s0), (r1, s1), (r2, s2), (r3, s3)),
                 (t0, t1, t2, t3), src_hbm, dst_hbm, w, sidx, didx,
                 src_hbm.shape[1])
  plsc.subcore_barrier()

  @pl.when(c == 0)
  def _():
    pltpu.sync_copy(acc.at[pl.ds(rbase, ROWS_PER_SUB)],
                    outa_hbm.at[pl.ds(rbase, ROWS_PER_SUB)])

  @pl.when(c == 1)
  def _():
    pltpu.sync_copy(acc.at[pl.ds(rbase, ROWS_PER_SUB)],
                    outb_hbm.at[pl.ds(rbase, ROWS_PER_SUB)])


@jax.jit
def _sc_agg1(xs, zeros, src, dst):
  return pl.kernel(
      _agg1_body,
      out_type=[jax.ShapeDtypeStruct((NP, D_IN), _f32),
                jax.ShapeDtypeStruct((NP, D_IN), _f32)],
      mesh=_mesh,
      scratch_types=[
          pltpu.VMEM_SHARED((NP, D_IN), _f32),
          pltpu.VMEM((CHUNK, D_IN), _f32),
          pltpu.VMEM((CHUNK, D_IN), _f32),
          pltpu.VMEM((CHUNK, D_IN), _f32),
          pltpu.VMEM((CHUNK, D_IN), _f32),
          pltpu.VMEM((IDXB, CHUNK), jnp.int32),
          pltpu.VMEM((IDXB, CHUNK), jnp.int32),
      ] + [pltpu.SemaphoreType.DMA] * 8,
  )(xs, zeros, src, dst)


def _agg2_body(ha_hbm, hb_hbm, src_hbm, dst_hbm, outa_hbm, outb_hbm,
               acc, r0, r1, r2, r3, sidx, didx, s0, s1, s2, s3,
               t0, t1, t2, t3):
  c = lax.axis_index("c")
  s = lax.axis_index("s")
  rbase = s * ROWS_PER_SUB

  def run(table_hbm, out_hbm):
    pltpu.sync_copy(table_hbm.at[pl.ds(rbase, ROWS_PER_SUB)],
                    acc.at[pl.ds(rbase, ROWS_PER_SUB)])
    plsc.subcore_barrier()
    _agg_pipelined(table_hbm, acc, ((r0, s0), (r1, s1), (r2, s2), (r3, s3)),
                   (t0, t1, t2, t3), src_hbm, dst_hbm, s, sidx, didx,
                   src_hbm.shape[1])
    plsc.subcore_barrier()
    pltpu.sync_copy(acc.at[pl.ds(rbase, ROWS_PER_SUB)],
                    out_hbm.at[pl.ds(rbase, ROWS_PER_SUB)])

  @pl.when(c == 0)
  def _():
    run(ha_hbm, outa_hbm)

  @pl.when(c == 1)
  def _():
    run(hb_hbm, outb_hbm)


@jax.jit
def _sc_agg2(ha, hb, src, dst):
  return pl.kernel(
      _agg2_body,
      out_type=[jax.ShapeDtypeStruct((NP, 128), _f32),
                jax.ShapeDtypeStruct((NP, 128), _f32)],
      mesh=_mesh,
      scratch_types=[
          pltpu.VMEM_SHARED((NP, 128), _f32),
          pltpu.VMEM((CHUNK, 128), _f32),
          pltpu.VMEM((CHUNK, 128), _f32),
          pltpu.VMEM((CHUNK, 128), _f32),
          pltpu.VMEM((CHUNK, 128), _f32),
          pltpu.VMEM((IDXB, CHUNK), jnp.int32),
          pltpu.VMEM((IDXB, CHUNK), jnp.int32),
      ] + [pltpu.SemaphoreType.DMA] * 8,
  )(ha, hb, src, dst)


def _tc1_body(dega_ref, degb_ref, x_ref, dinv_ref, xs_ref):
  i = pl.program_id(0)
  deg = dega_ref[...] + degb_ref[...] + 1.0
  dinv = lax.rsqrt(deg)
  row = i * BR + lax.broadcasted_iota(jnp.int32, (BR, 1), 0)
  valid = row < N
  dinv_ref[...] = dinv
  xs_ref[...] = jnp.where(valid, x_ref[...] * dinv, 0.0)


@jax.jit
def _tc_scale(dega, degb, x):
  return pl.pallas_call(
      _tc1_body,
      grid=(NBLK,),
      in_specs=[
          pl.BlockSpec((BR, 1), lambda i: (i, 0)),
          pl.BlockSpec((BR, 1), lambda i: (i, 0)),
          pl.BlockSpec((BR, D_IN), lambda i: (i, 0)),
      ],
      out_specs=[
          pl.BlockSpec((BR, 1), lambda i: (i, 0)),
          pl.BlockSpec((BR, D_IN), lambda i: (i, 0)),
      ],
      out_shape=[
          jax.ShapeDtypeStruct((NP, 1), _f32),
          jax.ShapeDtypeStruct((NP, D_IN), _f32),
      ],
  )(dega, degb, x)


def _tc2_body(p0_ref, p1_ref, dinv_ref, w1_ref, b1_ref, ha_ref, hb_ref):
  agg = p0_ref[...] + p1_ref[...]
  z = _dot(agg, w1_ref[...])
  dinv = dinv_ref[...]
  hs = jnp.maximum(z * dinv + b1_ref[...], 0.0) * dinv
  ha_ref[...] = hs[:, :128]
  hb_ref[...] = hs[:, 128:]


@jax.jit
def _tc_layer1(p0, p1, dinv, w1, b1):
  return pl.pallas_call(
      _tc2_body,
      grid=(NBLK,),
      in_specs=[
          pl.BlockSpec((BR, D_IN), lambda i: (i, 0)),
          pl.BlockSpec((BR, D_IN), lambda i: (i, 0)),
          pl.BlockSpec((BR, 1), lambda i: (i, 0)),
          pl.BlockSpec((D_IN, D_H), lambda i: (0, 0)),
          pl.BlockSpec((1, D_H), lambda i: (0, 0)),
      ],
      out_specs=[
          pl.BlockSpec((BR, 128), lambda i: (i, 0)),
          pl.BlockSpec((BR, 128), lambda i: (i, 0)),
      ],
      out_shape=[
          jax.ShapeDtypeStruct((NP, 128), _f32),
          jax.ShapeDtypeStruct((NP, 128), _f32),
      ],
  )(p0, p1, dinv, w1, b1)


def _tc3_body(qa_ref, qb_ref, dinv_ref, w2_ref, b2_ref, batch_ref,
              wfc_ref, bfc_ref, out_ref, pooled_acc, cnt_acc):
  i = pl.program_id(0)

  @pl.when(i == 0)
  def _():
    pooled_acc[...] = jnp.zeros_like(pooled_acc)
    cnt_acc[...] = jnp.zeros_like(cnt_acc)

  z = _dot(qa_ref[...], w2_ref[:128, :]) + _dot(qb_ref[...], w2_ref[128:, :])
  dinv = dinv_ref[...]
  h2 = jnp.maximum(z * dinv + b2_ref[...], 0.0)
  gids = lax.broadcasted_iota(jnp.int32, (G, BR), 0)
  oht = (batch_ref[0] == gids).astype(_f32)
  pooled_acc[...] += _dot(oht, h2)
  cnt_acc[...] += lax.dot_general(
      oht, jnp.ones((BR, 1), _f32), (((1,), (0,)), ((), ())),
      precision=_HIGH, preferred_element_type=_f32)

  @pl.when(i == NBLK - 1)
  def _():
    pooled = pooled_acc[...] / jnp.maximum(cnt_acc[...], 1.0)
    out_ref[...] = _dot(pooled, wfc_ref[...]) + bfc_ref[...]


@jax.jit
def _tc_layer2_pool(qa, qb, dinv, w2, b2, batch_rows, wfc, bfc):
  return pl.pallas_call(
      _tc3_body,
      grid=(NBLK,),
      in_specs=[
          pl.BlockSpec((BR, 128), lambda i: (i, 0)),
          pl.BlockSpec((BR, 128), lambda i: (i, 0)),
          pl.BlockSpec((BR, 1), lambda i: (i, 0)),
          pl.BlockSpec((D_H, D_H), lambda i: (0, 0)),
          pl.BlockSpec((1, D_H), lambda i: (0, 0)),
          pl.BlockSpec((1, 1, BR), lambda i: (i, 0, 0)),
          pl.BlockSpec((D_H, D_OUT), lambda i: (0, 0)),
          pl.BlockSpec((1, D_OUT), lambda i: (0, 0)),
      ],
      out_specs=pl.BlockSpec((G, D_OUT), lambda i: (0, 0)),
      out_shape=jax.ShapeDtypeStruct((G, D_OUT), _f32),
      scratch_shapes=[
          pltpu.VMEM((G, D_H), _f32),
          pltpu.VMEM((G, 1), _f32),
      ],
  )(qa, qb, dinv, w2, b2, batch_rows, wfc, bfc)


def _pad_chunk_edges(a, nworkers, pad_rows):
  per = E // nworkers
  npad = (-per) % (CHUNK * IDXB)
  a2 = a.reshape(nworkers, per)
  pad = jnp.broadcast_to(pad_rows[None, :npad], (nworkers, npad))
  return jnp.concatenate([a2, pad], axis=1).reshape(
      nworkers, (per + npad) // CHUNK, CHUNK)


def kernel(x, edge_index, batch, W1, b1, W2, b2, Wfc, bfc):
  src = edge_index[0]
  dst = edge_index[1]
  zeros = jnp.zeros((NP, D_IN), _f32)
  batch_rows = jnp.concatenate(
      [batch.astype(jnp.int32), jnp.full((NP - N,), G, jnp.int32)]
  ).reshape(NBLK, 1, BR)

  ar = jnp.arange(2048, dtype=jnp.int32)
  pad_src = (ar * 19) % N
  pad_dst = N + (ar % (NP - N))
  src1 = _pad_chunk_edges(src, 32, pad_src)
  dst1 = _pad_chunk_edges(dst, 32, pad_dst)
  src2 = _pad_chunk_edges(src, 16, pad_src)
  dst2 = _pad_chunk_edges(dst, 16, pad_dst)

  dega, degb = _sc_degree(dst)
  dinv, xs = _tc_scale(dega.reshape(NP, 1), degb.reshape(NP, 1), x)
  p0, p1 = _sc_agg1(xs, zeros, src1, dst1)
  ha, hb = _tc_layer1(p0, p1, dinv, W1, b1.reshape(1, D_H))
  qa, qb = _sc_agg2(ha, hb, src2, dst2)
  return _tc_layer2_pool(qa, qb, dinv, W2, b2.reshape(1, D_H),
                         batch_rows, Wfc, bfc.reshape(1, D_OUT))
